# Initial kernel scaffold; baseline (speedup 1.0000x reference)
import jax
import jax.numpy as jnp
from jax.experimental import pallas as pl


def kernel(x, table):
    raise NotImplementedError("write your pallas kernel here")


# speedup vs baseline: 1.7977x; 1.7977x over previous
import functools

import jax
import jax.numpy as jnp
from jax import lax
from jax.experimental import pallas as pl
from jax.experimental.pallas import tpu as pltpu
from jax.experimental.pallas import tpu_sc as plsc

_NC = 2
_NS = 16
_NW = _NC * _NS
_CH = 512


def _gather_body(x_hbm, table_hbm, out_hbm, idx_v, rows_v, sem):
    n = out_hbm.shape[0]
    per_w = n // _NW
    nchunk = per_w // _CH
    wid = lax.axis_index("s") * _NC + lax.axis_index("c")
    base = wid * per_w

    def chunk(i, carry):
        off = base + i * _CH
        pltpu.sync_copy(x_hbm.at[pl.ds(off, _CH)], idx_v)
        pltpu.async_copy(table_hbm.at[idx_v], rows_v, sem).wait()
        pltpu.sync_copy(rows_v, out_hbm.at[pl.ds(off, _CH)])
        return carry

    lax.fori_loop(0, nchunk, chunk, 0)


def kernel(x, table):
    b, h = x.shape
    n = b * h
    dim = table.shape[1]
    xf = x.reshape(n)
    mesh = plsc.VectorSubcoreMesh(core_axis_name="c", subcore_axis_name="s")
    out = pl.kernel(
        _gather_body,
        out_type=jax.ShapeDtypeStruct((n, dim), table.dtype),
        mesh=mesh,
        scratch_types=[
            pltpu.VMEM((_CH,), jnp.int32),
            pltpu.VMEM((_CH, dim), jnp.float32),
            pltpu.SemaphoreType.DMA,
        ],
        compiler_params=pltpu.CompilerParams(use_tc_tiling_on_sc=False),
    )(xf, table)
    return out.reshape(b, h, dim)


# speedup vs baseline: 1.8749x; 1.0429x over previous
import jax
import jax.numpy as jnp
from jax import lax
from jax.experimental import pallas as pl
from jax.experimental.pallas import tpu as pltpu
from jax.experimental.pallas import tpu_sc as plsc

_NC = 2
_NS = 16
_NW = _NC * _NS
_CH = 512


def _gather_body(x_hbm, table_hbm, out_hbm, idx_all, rows_v, sem_g, sem_w):
    n = out_hbm.shape[0]
    per_w = n // _NW
    nchunk = per_w // _CH
    wid = lax.axis_index("s") * _NC + lax.axis_index("c")
    base = wid * per_w

    def idx_slice(i):
        return idx_all.at[pl.ds(i * _CH, _CH)]

    def out_slice(i):
        return out_hbm.at[pl.ds(base + i * _CH, _CH)]

    pltpu.sync_copy(x_hbm.at[pl.ds(base, per_w)], idx_all)

    pltpu.async_copy(table_hbm.at[idx_slice(0)], rows_v.at[0], sem_g.at[0])

    def step(g, carry):
        for b in range(2):
            i = g * 2 + b
            nxt = 1 - b

            @pl.when(i + 1 < nchunk)
            def _start_next():
                @pl.when(i >= 1)
                def _drain_prev():
                    pltpu.make_async_copy(
                        rows_v.at[nxt], out_slice(i - 1), sem_w.at[nxt]
                    ).wait()

                pltpu.async_copy(
                    table_hbm.at[idx_slice(i + 1)], rows_v.at[nxt], sem_g.at[nxt]
                )

            pltpu.make_async_copy(
                table_hbm.at[idx_slice(i)], rows_v.at[b], sem_g.at[b]
            ).wait()
            pltpu.async_copy(rows_v.at[b], out_slice(i), sem_w.at[b])
        return carry

    lax.fori_loop(0, nchunk // 2, step, 0)

    pltpu.make_async_copy(rows_v.at[0], out_slice(nchunk - 2), sem_w.at[0]).wait()
    pltpu.make_async_copy(rows_v.at[1], out_slice(nchunk - 1), sem_w.at[1]).wait()


def kernel(x, table):
    b, h = x.shape
    n = b * h
    dim = table.shape[1]
    xf = x.reshape(n)
    mesh = plsc.VectorSubcoreMesh(core_axis_name="c", subcore_axis_name="s")
    out = pl.kernel(
        _gather_body,
        out_type=jax.ShapeDtypeStruct((n, dim), table.dtype),
        mesh=mesh,
        scratch_types=[
            pltpu.VMEM((n // _NW,), jnp.int32),
            pltpu.VMEM((2, _CH, dim), jnp.float32),
            pltpu.SemaphoreType.DMA((2,)),
            pltpu.SemaphoreType.DMA((2,)),
        ],
        compiler_params=pltpu.CompilerParams(use_tc_tiling_on_sc=False),
    )(xf, table)
    return out.reshape(b, h, dim)
